# baseline (speedup 1.0000x reference)
import functools

import jax
import jax.numpy as jnp
import numpy as np
from jax.experimental import pallas as pl
from jax.experimental.pallas import tpu as pltpu

BS_ = 4096
D_ = 128
NUM_CLS_ = 1024
K_ = 10
MARGIN_ = 0.1
RB_ = 128
NB_ = BS_ // RB_
TOPK_ = K_ + 1
WIN_ = 512
LMAX_ = WIN_ - RB_ - 191


def _w_of_pos(p):
    return 1.0 / float(np.log2(p + 1.0)) + 1.0


def _sim_block(x, y, sq_c, sq_r):
    g = jax.lax.dot_general(
        x, y, (((1,), (1,)), ((), ())),
        preferred_element_type=jnp.float32)
    d2 = jnp.maximum(sq_c + sq_r - 2.0 * g, 0.0)
    return jnp.sqrt(d2 + 1e-12)


def _pos_term_static(sim_hat, ks):
    neg_inf = jnp.float32(-jnp.inf)
    work = sim_hat
    acc = jnp.zeros((RB_, 1), jnp.float32)
    for p in range(1, TOPK_ + 1):
        m = jnp.max(work, axis=1, keepdims=True)
        work = jnp.where(work == m, neg_inf, work)
        acc += jnp.where(float(p) <= ks, m * _w_of_pos(p), 0.0)
    return acc


def _neg_term_static(workm, sim_hat, ks):
    neg_inf = jnp.float32(-jnp.inf)
    acc = jnp.zeros((RB_, 1), jnp.float32)
    for q in range(TOPK_):
        v = jnp.max(workm, axis=1, keepdims=True)
        workm = jnp.where(workm == v, neg_inf, workm)
        cnt_gt = jnp.sum((sim_hat > v).astype(jnp.float32), axis=1,
                         keepdims=True)
        w = 1.0 / jnp.log2(2.0 + cnt_gt) + 1.0
        acc += jnp.where(float(q) < ks, v * w, 0.0)
    return acc


def _accum_out(i, blk, out_ref):
    @pl.when(i == 0)
    def _():
        out_ref[...] = blk

    @pl.when(i > 0)
    def _():
        out_ref[...] += blk


def _full_kernel(xblk_ref, lab_col_ref, sq_col_ref, yall_ref, lab_row_ref,
                 sq_row_ref, out_ref):
    i = pl.program_id(0)
    dist = _sim_block(xblk_ref[...], yall_ref[...], sq_col_ref[...],
                      sq_row_ref[...])
    match = lab_col_ref[...] == lab_row_ref[...]
    sim_hat = jnp.where(match, -dist, MARGIN_ - dist)

    pos_num = jnp.sum(match.astype(jnp.float32), axis=1, keepdims=True)
    ks = jnp.minimum(pos_num, float(TOPK_))

    workm = jnp.where(match, sim_hat, jnp.float32(-jnp.inf))
    blk = jnp.sum(_pos_term_static(sim_hat, ks)
                  - _neg_term_static(workm, sim_hat, ks)).reshape(1, 1)
    _accum_out(i, blk, out_ref)


def _band_kernel(xblk_ref, lab_col_ref, sq_col_ref, pos_ref, yall_ref,
                 lab_row_ref, sq_row_ref, ywin_ref, lab_win_ref, sq_win_ref,
                 out_ref, work_ref, workm_ref, acc_ref):
    i = pl.program_id(0)
    x = xblk_ref[...]
    lab_c = lab_col_ref[...]
    sq_c = sq_col_ref[...]
    dist = _sim_block(x, yall_ref[...], sq_c, sq_row_ref[...])
    match = lab_c == lab_row_ref[...]
    sim_hat = jnp.where(match, -dist, MARGIN_ - dist)

    ks = jnp.minimum(pos_ref[...], float(TOPK_))
    mk = jnp.max(ks).astype(jnp.int32)

    distw = _sim_block(x, ywin_ref[0], sq_c, sq_win_ref[0])

    neg_inf = jnp.float32(-jnp.inf)
    work_ref[...] = sim_hat
    workm_ref[...] = jnp.where(lab_c == lab_win_ref[0], -distw, neg_inf)
    acc_ref[...] = jnp.zeros((RB_, 1), jnp.float32)

    for t in range(TOPK_):
        @pl.when(mk >= t + 1)
        def _():
            p = t + 1
            work = work_ref[...]
            m = jnp.max(work, axis=1, keepdims=True)
            work_ref[...] = jnp.where(work == m, neg_inf, work)

            workm = workm_ref[...]
            v = jnp.max(workm, axis=1, keepdims=True)
            workm_ref[...] = jnp.where(workm == v, neg_inf, workm)
            cnt_gt = jnp.sum((sim_hat > v).astype(jnp.float32), axis=1,
                             keepdims=True)
            w = 1.0 / jnp.log2(2.0 + cnt_gt) + 1.0

            acc_ref[...] += (jnp.where(float(p) <= ks, m * _w_of_pos(p), 0.0)
                             - jnp.where(float(t) < ks, v * w, 0.0))

    blk = jnp.sum(acc_ref[...]).reshape(1, 1)
    _accum_out(i, blk, out_ref)


def _run_full(x, lab_col, sq_col, lab_row, sq_row, pos_col):
    del pos_col
    return pl.pallas_call(
        _full_kernel,
        grid=(NB_,),
        in_specs=[
            pl.BlockSpec((RB_, D_), lambda i: (i, 0)),
            pl.BlockSpec((RB_, 1), lambda i: (i, 0)),
            pl.BlockSpec((RB_, 1), lambda i: (i, 0)),
            pl.BlockSpec((BS_, D_), lambda i: (0, 0)),
            pl.BlockSpec((1, BS_), lambda i: (0, 0)),
            pl.BlockSpec((1, BS_), lambda i: (0, 0)),
        ],
        out_specs=pl.BlockSpec((1, 1), lambda i: (0, 0)),
        out_shape=jax.ShapeDtypeStruct((1, 1), jnp.float32),
    )(x, lab_col, sq_col, x, lab_row, sq_row)


def _run_band(x, lab_col, sq_col, lab_row, sq_row, pos_col):
    starts = [min(max(i * RB_ - 192, 0), BS_ - WIN_) for i in range(NB_)]
    ywin = jnp.stack([jax.lax.slice(x, (s, 0), (s + WIN_, D_))
                      for s in starts])
    labw = jnp.stack([jax.lax.slice(lab_row, (0, s), (1, s + WIN_))
                      for s in starts])
    sqw = jnp.stack([jax.lax.slice(sq_row, (0, s), (1, s + WIN_))
                     for s in starts])
    return pl.pallas_call(
        _band_kernel,
        grid=(NB_,),
        in_specs=[
            pl.BlockSpec((RB_, D_), lambda i: (i, 0)),
            pl.BlockSpec((RB_, 1), lambda i: (i, 0)),
            pl.BlockSpec((RB_, 1), lambda i: (i, 0)),
            pl.BlockSpec((RB_, 1), lambda i: (i, 0)),
            pl.BlockSpec((BS_, D_), lambda i: (0, 0)),
            pl.BlockSpec((1, BS_), lambda i: (0, 0)),
            pl.BlockSpec((1, BS_), lambda i: (0, 0)),
            pl.BlockSpec((1, WIN_, D_), lambda i: (i, 0, 0)),
            pl.BlockSpec((1, 1, WIN_), lambda i: (i, 0, 0)),
            pl.BlockSpec((1, 1, WIN_), lambda i: (i, 0, 0)),
        ],
        out_specs=pl.BlockSpec((1, 1), lambda i: (0, 0)),
        out_shape=jax.ShapeDtypeStruct((1, 1), jnp.float32),
        scratch_shapes=[
            pltpu.VMEM((RB_, BS_), jnp.float32),
            pltpu.VMEM((RB_, WIN_), jnp.float32),
            pltpu.VMEM((RB_, 1), jnp.float32),
        ],
    )(x, lab_col, sq_col, pos_col, x, lab_row, sq_row, ywin, labw, sqw)


@jax.jit
def kernel(batch_reprs, batch_labels):
    counts = jnp.bincount(batch_labels, length=NUM_CLS_)
    pos_all = counts[batch_labels].astype(jnp.int32)
    order = jnp.argsort((BS_ - pos_all) * NUM_CLS_ + batch_labels)
    labs = batch_labels[order]
    x = batch_reprs[order]
    pos_num = pos_all[order].astype(jnp.float32)
    l_max = jnp.max(counts)

    sq = jnp.sum(x * x, axis=1)
    args = (x, labs.reshape(BS_, 1), sq.reshape(BS_, 1),
            labs.reshape(1, BS_), sq.reshape(1, BS_),
            pos_num.reshape(BS_, 1))
    out = jax.lax.cond(l_max <= LMAX_, _run_band, _run_full, *args)
    return out[0, 0]


# speedup vs baseline: 1.1243x; 1.1243x over previous
import jax
import jax.numpy as jnp
import numpy as np
from jax.experimental import pallas as pl
from jax.experimental.pallas import tpu as pltpu

BS_ = 4096
D_ = 128
NUM_CLS_ = 1024
K_ = 10
MARGIN_ = 0.1
RB_ = 256
NB_ = BS_ // RB_
TOPK_ = K_ + 1
WIN_ = 512
LMAX_ = WIN_ - RB_ - 127


def _w_of_pos(p):
    return 1.0 / float(np.log2(p + 1.0)) + 1.0


def _sim_block(x, y, sq_c, sq_r):
    g = jax.lax.dot_general(
        x, y, (((1,), (1,)), ((), ())),
        preferred_element_type=jnp.float32)
    d2 = jnp.maximum(sq_c + sq_r - 2.0 * g, 0.0)
    return jnp.sqrt(d2 + 1e-12)


def _pos_term_static(sim_hat, ks):
    neg_inf = jnp.float32(-jnp.inf)
    work = sim_hat
    acc = jnp.zeros((RB_, 1), jnp.float32)
    for p in range(1, TOPK_ + 1):
        m = jnp.max(work, axis=1, keepdims=True)
        work = jnp.where(work == m, neg_inf, work)
        acc += jnp.where(float(p) <= ks, m * _w_of_pos(p), 0.0)
    return acc


def _neg_term_static(workm, sim_hat, ks):
    neg_inf = jnp.float32(-jnp.inf)
    acc = jnp.zeros((RB_, 1), jnp.float32)
    for q in range(TOPK_):
        v = jnp.max(workm, axis=1, keepdims=True)
        workm = jnp.where(workm == v, neg_inf, workm)
        cnt_gt = jnp.sum((sim_hat > v).astype(jnp.float32), axis=1,
                         keepdims=True)
        w = 1.0 / jnp.log2(2.0 + cnt_gt) + 1.0
        acc += jnp.where(float(q) < ks, v * w, 0.0)
    return acc


def _accum_out(i, blk, out_ref):
    @pl.when(i == 0)
    def _():
        out_ref[...] = blk

    @pl.when(i > 0)
    def _():
        out_ref[...] += blk


def _full_kernel(xblk_ref, lab_col_ref, sq_col_ref, yall_ref, lab_row_ref,
                 sq_row_ref, out_ref):
    i = pl.program_id(0)
    dist = _sim_block(xblk_ref[...], yall_ref[...], sq_col_ref[...],
                      sq_row_ref[...])
    match = lab_col_ref[...] == lab_row_ref[...]
    sim_hat = jnp.where(match, -dist, MARGIN_ - dist)

    pos_num = jnp.sum(match.astype(jnp.float32), axis=1, keepdims=True)
    ks = jnp.minimum(pos_num, float(TOPK_))

    workm = jnp.where(match, sim_hat, jnp.float32(-jnp.inf))
    blk = jnp.sum(_pos_term_static(sim_hat, ks)
                  - _neg_term_static(workm, sim_hat, ks)).reshape(1, 1)
    _accum_out(i, blk, out_ref)


def _band_kernel(xblk_ref, lab_col_ref, sq_col_ref, pos_ref, yall_ref,
                 lab_row_ref, sq_row_ref, ywin_ref, lab_win_ref, sq_win_ref,
                 out_ref, work_ref, workm_ref, acc_ref):
    i = pl.program_id(0)
    x = xblk_ref[...]
    lab_c = lab_col_ref[...]
    sq_c = sq_col_ref[...]
    dist = _sim_block(x, yall_ref[...], sq_c, sq_row_ref[...])
    match = lab_c == lab_row_ref[...]
    sim_hat = jnp.where(match, -dist, MARGIN_ - dist)

    ks = jnp.minimum(pos_ref[...], float(TOPK_))
    mk = jnp.max(ks).astype(jnp.int32)

    distw = _sim_block(x, ywin_ref[0], sq_c, sq_win_ref[0])

    neg_inf = jnp.float32(-jnp.inf)
    work_ref[...] = sim_hat
    workm_ref[...] = jnp.where(lab_c == lab_win_ref[0], -distw, neg_inf)
    acc_ref[...] = jnp.zeros((RB_, 1), jnp.float32)

    for t in range(TOPK_):
        @pl.when(mk >= t + 1)
        def _():
            p = t + 1
            work = work_ref[...]
            m = jnp.max(work, axis=1, keepdims=True)
            work_ref[...] = jnp.where(work == m, neg_inf, work)

            workm = workm_ref[...]
            v = jnp.max(workm, axis=1, keepdims=True)
            workm_ref[...] = jnp.where(workm == v, neg_inf, workm)
            cnt_gt = jnp.sum((sim_hat > v).astype(jnp.float32), axis=1,
                             keepdims=True)
            w = 1.0 / jnp.log2(2.0 + cnt_gt) + 1.0

            acc_ref[...] += (jnp.where(float(p) <= ks, m * _w_of_pos(p), 0.0)
                             - jnp.where(float(t) < ks, v * w, 0.0))

    blk = jnp.sum(acc_ref[...]).reshape(1, 1)
    _accum_out(i, blk, out_ref)


def _run_full(x, lab_col, sq_col, lab_row, sq_row, pos_col):
    del pos_col
    return pl.pallas_call(
        _full_kernel,
        grid=(NB_,),
        in_specs=[
            pl.BlockSpec((RB_, D_), lambda i: (i, 0)),
            pl.BlockSpec((RB_, 1), lambda i: (i, 0)),
            pl.BlockSpec((RB_, 1), lambda i: (i, 0)),
            pl.BlockSpec((BS_, D_), lambda i: (0, 0)),
            pl.BlockSpec((1, BS_), lambda i: (0, 0)),
            pl.BlockSpec((1, BS_), lambda i: (0, 0)),
        ],
        out_specs=pl.BlockSpec((1, 1), lambda i: (0, 0)),
        out_shape=jax.ShapeDtypeStruct((1, 1), jnp.float32),
    )(x, lab_col, sq_col, x, lab_row, sq_row)


def _run_band(x, lab_col, sq_col, lab_row, sq_row, pos_col):
    starts = [min(max(i * RB_ - 128, 0), BS_ - WIN_) for i in range(NB_)]
    ywin = jnp.stack([jax.lax.slice(x, (s, 0), (s + WIN_, D_))
                      for s in starts])
    labw = jnp.stack([jax.lax.slice(lab_row, (0, s), (1, s + WIN_))
                      for s in starts])
    sqw = jnp.stack([jax.lax.slice(sq_row, (0, s), (1, s + WIN_))
                     for s in starts])
    return pl.pallas_call(
        _band_kernel,
        grid=(NB_,),
        in_specs=[
            pl.BlockSpec((RB_, D_), lambda i: (i, 0)),
            pl.BlockSpec((RB_, 1), lambda i: (i, 0)),
            pl.BlockSpec((RB_, 1), lambda i: (i, 0)),
            pl.BlockSpec((RB_, 1), lambda i: (i, 0)),
            pl.BlockSpec((BS_, D_), lambda i: (0, 0)),
            pl.BlockSpec((1, BS_), lambda i: (0, 0)),
            pl.BlockSpec((1, BS_), lambda i: (0, 0)),
            pl.BlockSpec((1, WIN_, D_), lambda i: (i, 0, 0)),
            pl.BlockSpec((1, 1, WIN_), lambda i: (i, 0, 0)),
            pl.BlockSpec((1, 1, WIN_), lambda i: (i, 0, 0)),
        ],
        out_specs=pl.BlockSpec((1, 1), lambda i: (0, 0)),
        out_shape=jax.ShapeDtypeStruct((1, 1), jnp.float32),
        scratch_shapes=[
            pltpu.VMEM((RB_, BS_), jnp.float32),
            pltpu.VMEM((RB_, WIN_), jnp.float32),
            pltpu.VMEM((RB_, 1), jnp.float32),
        ],
    )(x, lab_col, sq_col, pos_col, x, lab_row, sq_row, ywin, labw, sqw)


@jax.jit
def kernel(batch_reprs, batch_labels):
    counts = jnp.bincount(batch_labels, length=NUM_CLS_)
    pos_all = counts[batch_labels].astype(jnp.int32)
    order = jnp.argsort((BS_ - pos_all) * NUM_CLS_ + batch_labels)
    labs = batch_labels[order]
    x = batch_reprs[order]
    pos_num = pos_all[order].astype(jnp.float32)
    l_max = jnp.max(counts)

    sq = jnp.sum(x * x, axis=1)
    args = (x, labs.reshape(BS_, 1), sq.reshape(BS_, 1),
            labs.reshape(1, BS_), sq.reshape(1, BS_),
            pos_num.reshape(BS_, 1))
    out = jax.lax.cond(l_max <= LMAX_, _run_band, _run_full, *args)
    return out[0, 0]
